# baseline (speedup 1.0000x reference)
import functools

import jax
import jax.numpy as jnp
from jax import lax
from jax.experimental import layout as jexp_layout
from jax.experimental import pallas as pl
from jax.experimental.pallas import tpu as pltpu
from jax.experimental.pallas import tpu_sc as plsc

B, L, D_BERT = 1024, 50, 768
ED = 32
N_TOKENS = B * L
D_OUT = D_BERT + 2 * ED
TROWS = 25000

CHUNK = 80
NCHUNK = 20
NBUF = 4
WBUF = 2


def _extract_chunk(rows_ref, rv, ext_ref, j, jm, wm, col0):
    for i0 in range(CHUNK // 16):
        row16 = lax.iota(jnp.int32, 16) + i0 * 16
        r16 = rv[pl.ds(j * CHUNK + i0 * 16, 16)]
        rm = r16 * ED
        for c in range(ED):
            v = plsc.load_gather(rows_ref.at[jm], [row16, rm + c])
            plsc.store_scatter(ext_ref.at[wm],
                               [row16, jnp.full((16,), col0 + c, jnp.int32)],
                               v)


def _sc_gather_body(t0_hbm, t1_hbm, q0_hbm, q1_hbm, r0_hbm, r1_hbm, g01_hbm,
                    qv0, qv1, rv0, rv1, rows0, rows1, ext,
                    sem0, sem1, semw, *, num_cores):
    wid = lax.axis_index("s") * num_cores + lax.axis_index("c")
    base = wid * NCHUNK * CHUNK
    pltpu.sync_copy(q0_hbm.at[pl.ds(base, NCHUNK * CHUNK)], qv0)
    pltpu.sync_copy(q1_hbm.at[pl.ds(base, NCHUNK * CHUNK)], qv1)
    pltpu.sync_copy(r0_hbm.at[pl.ds(base, NCHUNK * CHUNK)], rv0)
    pltpu.sync_copy(r1_hbm.at[pl.ds(base, NCHUNK * CHUNK)], rv1)

    def fire(t_hbm, qv, rows, sem, j):
        return pltpu.async_copy(
            t_hbm.at[qv.at[pl.ds(j * CHUNK, CHUNK)]], rows.at[j % NBUF], sem)

    def wait_gather(t_hbm, rows, sem, jm):
        pltpu.make_async_copy(t_hbm.at[pl.ds(0, CHUNK)], rows.at[jm],
                              sem).wait()

    def wait_write(wm):
        pltpu.make_async_copy(ext.at[wm], g01_hbm.at[pl.ds(base, CHUNK)],
                              semw).wait()

    for j in range(NBUF):
        fire(t0_hbm, qv0, rows0, sem0, j)
        fire(t1_hbm, qv1, rows1, sem1, j)

    def step(j, _):
        jm = j % NBUF
        wm = j % WBUF

        @pl.when(j >= WBUF)
        def _():
            wait_write(wm)

        wait_gather(t0_hbm, rows0, sem0, jm)
        _extract_chunk(rows0, rv0, ext, j, jm, wm, 0)
        wait_gather(t1_hbm, rows1, sem1, jm)
        _extract_chunk(rows1, rv1, ext, j, jm, wm, ED)
        pltpu.async_copy(ext.at[wm],
                         g01_hbm.at[pl.ds(base + j * CHUNK, CHUNK)], semw)

        @pl.when(j < NCHUNK - NBUF)
        def _():
            fire(t0_hbm, qv0, rows0, sem0, j + NBUF)
            fire(t1_hbm, qv1, rows1, sem1, j + NBUF)

        return _

    lax.fori_loop(0, NCHUNK, step, None)
    wait_write((NCHUNK - 2) % WBUF)
    wait_write((NCHUNK - 1) % WBUF)


def _sc_gather(t0r, t1r, q0, q1, r0, r1, *, interpret=False):
    try:
        num_cores = plsc.get_sparse_core_info().num_cores
    except ValueError:
        num_cores = 2
    mesh = plsc.VectorSubcoreMesh(core_axis_name="c", subcore_axis_name="s",
                                  num_cores=num_cores, num_subcores=16)
    body = functools.partial(_sc_gather_body, num_cores=num_cores)
    n_per_w = NCHUNK * CHUNK
    return pl.kernel(
        body,
        out_type=jax.ShapeDtypeStruct((N_TOKENS, 128), jnp.float32),
        mesh=mesh,
        scratch_types=[
            pltpu.VMEM((n_per_w,), jnp.int32),
            pltpu.VMEM((n_per_w,), jnp.int32),
            pltpu.VMEM((n_per_w,), jnp.int32),
            pltpu.VMEM((n_per_w,), jnp.int32),
            pltpu.VMEM((NBUF, CHUNK, 128), jnp.float32),
            pltpu.VMEM((NBUF, CHUNK, 128), jnp.float32),
            pltpu.VMEM((WBUF, CHUNK, 128), jnp.float32),
            pltpu.SemaphoreType.DMA,
            pltpu.SemaphoreType.DMA,
            pltpu.SemaphoreType.DMA,
        ],
        compiler_params=pltpu.CompilerParams(use_tc_tiling_on_sc=False,
                                             needs_layout_passes=False),
        interpret=interpret,
    )(t0r, t1r, q0, q1, r0, r1)


def _concat_body(bert_ref, g01_ref, out_ref):
    out_ref[0, :, :D_BERT] = bert_ref[0]
    out_ref[0, :, D_BERT:D_OUT] = g01_ref[:, :2 * ED]


def _tc_concat(bert_t, g01):
    return pl.pallas_call(
        _concat_body,
        grid=(L,),
        in_specs=[
            pl.BlockSpec((1, B, D_BERT), lambda i: (i, 0, 0)),
            pl.BlockSpec((B, 128), lambda i: (i, 0)),
        ],
        out_specs=pl.BlockSpec((1, B, D_OUT), lambda i: (i, 0, 0)),
        out_shape=jax.ShapeDtypeStruct((L, B, D_OUT), jnp.float32),
    )(bert_t, g01)


@jax.jit
def kernel(bert_embed, features0, features1, table0, table1):
    f0 = features0.astype(jnp.int32).T.reshape(N_TOKENS)
    f1 = features1.astype(jnp.int32).T.reshape(N_TOKENS)
    g01 = _sc_gather(table0.reshape(TROWS, 128), table1.reshape(TROWS, 128),
                     f0 >> 2, f1 >> 2, f0 & 3, f1 & 3)
    g3 = g01.reshape(L, B, 128)
    e0 = jnp.transpose(g3[:, :, :ED], (1, 0, 2))
    e1 = jnp.transpose(g3[:, :, ED:2 * ED], (1, 0, 2))
    return jnp.concatenate([bert_embed, e0, e1], axis=2)


# speedup vs baseline: 1.4366x; 1.4366x over previous
import functools

import jax
import jax.numpy as jnp
from jax import lax
from jax.experimental import layout as jexp_layout
from jax.experimental import pallas as pl
from jax.experimental.pallas import tpu as pltpu
from jax.experimental.pallas import tpu_sc as plsc

B, L, D_BERT = 1024, 50, 768
ED = 32
N_TOKENS = B * L
D_OUT = D_BERT + 2 * ED
TROWS = 25000

CHUNK = 80
NCHUNK = 20
NBUF = 4
WBUF = 2


def _extract_chunk(rows_ref, rv, ext_ref, j, jm, wm, col0):
    for i0 in range(CHUNK // 16):
        row16 = lax.iota(jnp.int32, 16) + i0 * 16
        r16 = rv[pl.ds(j * CHUNK + i0 * 16, 16)]
        rm = r16 * ED
        for c in range(ED):
            v = plsc.load_gather(rows_ref.at[jm], [row16, rm + c])
            plsc.store_scatter(ext_ref.at[wm],
                               [row16, jnp.full((16,), col0 + c, jnp.int32)],
                               v)


def _sc_gather_body(t0_hbm, t1_hbm, q0_hbm, q1_hbm, r0_hbm, r1_hbm, g01_hbm,
                    qv0, qv1, rv0, rv1, rows0, rows1, ext,
                    sem0, sem1, semw, *, num_cores):
    wid = lax.axis_index("s") * num_cores + lax.axis_index("c")
    base = wid * NCHUNK * CHUNK
    pltpu.sync_copy(q0_hbm.at[pl.ds(base, NCHUNK * CHUNK)], qv0)
    pltpu.sync_copy(q1_hbm.at[pl.ds(base, NCHUNK * CHUNK)], qv1)
    pltpu.sync_copy(r0_hbm.at[pl.ds(base, NCHUNK * CHUNK)], rv0)
    pltpu.sync_copy(r1_hbm.at[pl.ds(base, NCHUNK * CHUNK)], rv1)

    def fire(t_hbm, qv, rows, sem, j):
        return pltpu.async_copy(
            t_hbm.at[qv.at[pl.ds(j * CHUNK, CHUNK)]], rows.at[j % NBUF], sem)

    def wait_gather(t_hbm, rows, sem, jm):
        pltpu.make_async_copy(t_hbm.at[pl.ds(0, CHUNK)], rows.at[jm],
                              sem).wait()

    def wait_write(wm):
        pltpu.make_async_copy(ext.at[wm], g01_hbm.at[pl.ds(base, CHUNK)],
                              semw).wait()

    for j in range(NBUF):
        fire(t0_hbm, qv0, rows0, sem0, j)
        fire(t1_hbm, qv1, rows1, sem1, j)

    def step(j, _):
        jm = j % NBUF
        wm = j % WBUF

        @pl.when(j >= WBUF)
        def _():
            wait_write(wm)

        wait_gather(t0_hbm, rows0, sem0, jm)
        _extract_chunk(rows0, rv0, ext, j, jm, wm, 0)
        wait_gather(t1_hbm, rows1, sem1, jm)
        _extract_chunk(rows1, rv1, ext, j, jm, wm, ED)
        pltpu.async_copy(ext.at[wm],
                         g01_hbm.at[pl.ds(base + j * CHUNK, CHUNK)], semw)

        @pl.when(j < NCHUNK - NBUF)
        def _():
            fire(t0_hbm, qv0, rows0, sem0, j + NBUF)
            fire(t1_hbm, qv1, rows1, sem1, j + NBUF)

        return _

    lax.fori_loop(0, NCHUNK, step, None)
    wait_write((NCHUNK - 2) % WBUF)
    wait_write((NCHUNK - 1) % WBUF)


def _sc_gather(t0r, t1r, q0, q1, r0, r1, *, interpret=False):
    try:
        num_cores = plsc.get_sparse_core_info().num_cores
    except ValueError:
        num_cores = 2
    mesh = plsc.VectorSubcoreMesh(core_axis_name="c", subcore_axis_name="s",
                                  num_cores=num_cores, num_subcores=16)
    body = functools.partial(_sc_gather_body, num_cores=num_cores)
    n_per_w = NCHUNK * CHUNK
    return pl.kernel(
        body,
        out_type=jax.ShapeDtypeStruct((N_TOKENS, 128), jnp.float32),
        mesh=mesh,
        scratch_types=[
            pltpu.VMEM((n_per_w,), jnp.int32),
            pltpu.VMEM((n_per_w,), jnp.int32),
            pltpu.VMEM((n_per_w,), jnp.int32),
            pltpu.VMEM((n_per_w,), jnp.int32),
            pltpu.VMEM((NBUF, CHUNK, 128), jnp.float32),
            pltpu.VMEM((NBUF, CHUNK, 128), jnp.float32),
            pltpu.VMEM((WBUF, CHUNK, 128), jnp.float32),
            pltpu.SemaphoreType.DMA,
            pltpu.SemaphoreType.DMA,
            pltpu.SemaphoreType.DMA,
        ],
        compiler_params=pltpu.CompilerParams(use_tc_tiling_on_sc=False,
                                             needs_layout_passes=False),
        interpret=interpret,
    )(t0r, t1r, q0, q1, r0, r1)


def _concat_body(bert_ref, g01_ref, out_ref):
    out_ref[0, :D_BERT, :] = jnp.swapaxes(bert_ref[0], 0, 1)
    out_ref[0, D_BERT:D_OUT, :] = jnp.swapaxes(g01_ref[:, :2 * ED], 0, 1)


def _tc_concat(bert_t, g01):
    return pl.pallas_call(
        _concat_body,
        grid=(L,),
        in_specs=[
            pl.BlockSpec((1, B, D_BERT), lambda i: (i, 0, 0)),
            pl.BlockSpec((B, 128), lambda i: (i, 0)),
        ],
        out_specs=pl.BlockSpec((1, D_OUT, B), lambda i: (i, 0, 0)),
        out_shape=jax.ShapeDtypeStruct((L, D_OUT, B), jnp.float32),
    )(bert_t, g01)


@jax.jit
def kernel(bert_embed, features0, features1, table0, table1):
    f0 = features0.astype(jnp.int32).T.reshape(N_TOKENS)
    f1 = features1.astype(jnp.int32).T.reshape(N_TOKENS)
    g01 = _sc_gather(table0.reshape(TROWS, 128), table1.reshape(TROWS, 128),
                     f0 >> 2, f1 >> 2, f0 & 3, f1 & 3)
    bert_t = jnp.transpose(bert_embed, (1, 0, 2))
    out_t = _tc_concat(bert_t, g01)
    return jnp.transpose(out_t, (2, 0, 1))


# speedup vs baseline: 1.7903x; 1.2463x over previous
import functools

import jax
import jax.numpy as jnp
from jax import lax
from jax.experimental import layout as jexp_layout
from jax.experimental import pallas as pl
from jax.experimental.pallas import tpu as pltpu
from jax.experimental.pallas import tpu_sc as plsc

B, L, D_BERT = 1024, 50, 768
ED = 32
N_TOKENS = B * L
D_OUT = D_BERT + 2 * ED
TROWS = 25000

CHUNK = 80
NCHUNK = 20
NBUF = 4
WBUF = 2


def _extract_chunk(rows_ref, rv, ext_ref, j, jm, wm, col0):
    for i0 in range(CHUNK // 16):
        row16 = lax.iota(jnp.int32, 16) + i0 * 16
        r16 = rv[pl.ds(j * CHUNK + i0 * 16, 16)]
        rm = r16 * ED
        for c in range(ED):
            v = plsc.load_gather(rows_ref.at[jm], [row16, rm + c])
            plsc.store_scatter(ext_ref.at[wm],
                               [row16, jnp.full((16,), col0 + c, jnp.int32)],
                               v)


def _sc_gather_body(t0_hbm, t1_hbm, q0_hbm, q1_hbm, r0_hbm, r1_hbm, g01_hbm,
                    qv0, qv1, rv0, rv1, rows0, rows1, ext,
                    sem0, sem1, semw, *, num_cores):
    wid = lax.axis_index("s") * num_cores + lax.axis_index("c")
    base = wid * NCHUNK * CHUNK
    pltpu.sync_copy(q0_hbm.at[pl.ds(base, NCHUNK * CHUNK)], qv0)
    pltpu.sync_copy(q1_hbm.at[pl.ds(base, NCHUNK * CHUNK)], qv1)
    pltpu.sync_copy(r0_hbm.at[pl.ds(base, NCHUNK * CHUNK)], rv0)
    pltpu.sync_copy(r1_hbm.at[pl.ds(base, NCHUNK * CHUNK)], rv1)

    def fire(t_hbm, qv, rows, sem, j):
        return pltpu.async_copy(
            t_hbm.at[qv.at[pl.ds(j * CHUNK, CHUNK)]], rows.at[j % NBUF], sem)

    def wait_gather(t_hbm, rows, sem, jm):
        pltpu.make_async_copy(t_hbm.at[pl.ds(0, CHUNK)], rows.at[jm],
                              sem).wait()

    def wait_write(wm):
        pltpu.make_async_copy(ext.at[wm], g01_hbm.at[pl.ds(base, CHUNK)],
                              semw).wait()

    for j in range(NBUF):
        fire(t0_hbm, qv0, rows0, sem0, j)
        fire(t1_hbm, qv1, rows1, sem1, j)

    def step(j, _):
        jm = j % NBUF
        wm = j % WBUF

        @pl.when(j >= WBUF)
        def _():
            wait_write(wm)

        wait_gather(t0_hbm, rows0, sem0, jm)
        _extract_chunk(rows0, rv0, ext, j, jm, wm, 0)
        wait_gather(t1_hbm, rows1, sem1, jm)
        _extract_chunk(rows1, rv1, ext, j, jm, wm, ED)
        pltpu.async_copy(ext.at[wm],
                         g01_hbm.at[pl.ds(base + j * CHUNK, CHUNK)], semw)

        @pl.when(j < NCHUNK - NBUF)
        def _():
            fire(t0_hbm, qv0, rows0, sem0, j + NBUF)
            fire(t1_hbm, qv1, rows1, sem1, j + NBUF)

        return _

    lax.fori_loop(0, NCHUNK, step, None)
    wait_write((NCHUNK - 2) % WBUF)
    wait_write((NCHUNK - 1) % WBUF)


def _sc_gather(t0r, t1r, q0, q1, r0, r1, *, interpret=False):
    try:
        num_cores = plsc.get_sparse_core_info().num_cores
    except ValueError:
        num_cores = 2
    mesh = plsc.VectorSubcoreMesh(core_axis_name="c", subcore_axis_name="s",
                                  num_cores=num_cores, num_subcores=16)
    body = functools.partial(_sc_gather_body, num_cores=num_cores)
    n_per_w = NCHUNK * CHUNK
    return pl.kernel(
        body,
        out_type=jax.ShapeDtypeStruct((N_TOKENS, 128), jnp.float32),
        mesh=mesh,
        scratch_types=[
            pltpu.VMEM((n_per_w,), jnp.int32),
            pltpu.VMEM((n_per_w,), jnp.int32),
            pltpu.VMEM((n_per_w,), jnp.int32),
            pltpu.VMEM((n_per_w,), jnp.int32),
            pltpu.VMEM((NBUF, CHUNK, 128), jnp.float32),
            pltpu.VMEM((NBUF, CHUNK, 128), jnp.float32),
            pltpu.VMEM((WBUF, CHUNK, 128), jnp.float32),
            pltpu.SemaphoreType.DMA,
            pltpu.SemaphoreType.DMA,
            pltpu.SemaphoreType.DMA,
        ],
        compiler_params=pltpu.CompilerParams(use_tc_tiling_on_sc=False,
                                             needs_layout_passes=False),
        interpret=interpret,
    )(t0r, t1r, q0, q1, r0, r1)


def _bert_body(bert_ref, out_ref):
    out_ref[0, :, :] = jnp.swapaxes(bert_ref[0], 0, 1)


def _emb_body(_, g01_ref, out_ref):
    out_ref[0, :, :] = jnp.swapaxes(g01_ref[:, :2 * ED], 0, 1)


def _tc_concat(bert_t, g01):
    out_shape = jax.ShapeDtypeStruct((L, D_OUT, B), jnp.float32)
    partial = pl.pallas_call(
        _bert_body,
        grid=(L,),
        in_specs=[pl.BlockSpec((1, B, D_BERT), lambda i: (i, 0, 0))],
        out_specs=pl.BlockSpec((1, D_BERT, B), lambda i: (i, 0, 0)),
        out_shape=out_shape,
    )(bert_t)
    return pl.pallas_call(
        _emb_body,
        grid=(L,),
        in_specs=[
            pl.BlockSpec((1, 2 * ED, B), lambda i: (i, 12, 0)),
            pl.BlockSpec((B, 128), lambda i: (i, 0)),
        ],
        out_specs=pl.BlockSpec((1, 2 * ED, B), lambda i: (i, 12, 0)),
        out_shape=out_shape,
        input_output_aliases={0: 0},
    )(partial, g01)


@jax.jit
def kernel(bert_embed, features0, features1, table0, table1):
    f0 = features0.astype(jnp.int32).T.reshape(N_TOKENS)
    f1 = features1.astype(jnp.int32).T.reshape(N_TOKENS)
    g01 = _sc_gather(table0.reshape(TROWS, 128), table1.reshape(TROWS, 128),
                     f0 >> 2, f1 >> 2, f0 & 3, f1 & 3)
    bert_t = jnp.transpose(bert_embed, (1, 0, 2))
    out_t = _tc_concat(bert_t, g01)
    return jnp.transpose(out_t, (2, 0, 1))
